# baseline (speedup 1.0000x reference)
import functools

import jax
import jax.numpy as jnp
from jax import lax
from jax.experimental import pallas as pl
from jax.experimental.pallas import tpu as pltpu
from jax.experimental.pallas import tpu_sc as plsc

_ALPHA = 3.0
_K = 20


def _gather_rows_sc(table, idx):
    info = plsc.get_sparse_core_info()
    nc, ns = info.num_cores, info.num_subcores
    nw = nc * ns
    b, d = idx.shape[0], table.shape[1]
    b_per_w = b // nw
    mesh = plsc.VectorSubcoreMesh(core_axis_name="c", subcore_axis_name="s")

    @functools.partial(
        pl.kernel,
        mesh=mesh,
        compiler_params=pltpu.CompilerParams(use_tc_tiling_on_sc=False),
        out_type=jax.ShapeDtypeStruct((b, d), jnp.float32),
        scratch_types=[
            pltpu.VMEM((b_per_w,), jnp.int32),
            pltpu.VMEM((b_per_w, d), jnp.float32),
            pltpu.SemaphoreType.DMA,
        ],
    )
    def gather_kernel(table_hbm, idx_hbm, out_hbm, idx_v, rows_v, sem):
        wid = lax.axis_index("s") * nc + lax.axis_index("c")
        base = wid * b_per_w
        pltpu.sync_copy(idx_hbm.at[pl.ds(base, b_per_w)], idx_v)
        pltpu.async_copy(table_hbm.at[idx_v], rows_v, sem).wait()
        pltpu.sync_copy(rows_v, out_hbm.at[pl.ds(base, b_per_w)])

    return gather_kernel(table, idx)


def _h_body(x_ref, w1_ref, b1_ref, he_ref, w2_ref, b2_ref, h_ref):
    dn = (((1,), (1,)), ((), ()))
    z1 = lax.dot_general(x_ref[...], w1_ref[...], dn,
                         preferred_element_type=jnp.float32)
    nv1 = jnp.tanh(_ALPHA * (z1 + b1_ref[...]))
    z2 = lax.dot_general(he_ref[...], w2_ref[...], dn,
                         preferred_element_type=jnp.float32)
    nv2 = jnp.tanh(_ALPHA * (z2 + b2_ref[...]))
    h0 = lax.dot_general(nv1, nv2, dn, preferred_element_type=jnp.float32)
    h_ref[...] = jnp.maximum(jnp.tanh(_ALPHA * h0), 0.0)


def _fused_body(hb1_ref, hb2_ref, hall_ref, out_ref, a16_ref, w16_ref,
                t_scr, c_scr):
    b, blk = a16_ref.shape
    hb_ref = hb1_ref
    adj_t = lax.dot_general(hall_ref[...], hb_ref[...],
                            (((1,), (1,)), ((), ())),
                            preferred_element_type=jnp.float32)
    keys = lax.bitcast_convert_type(adj_t, jnp.int32)
    a16 = ((keys >> 15) - 32768).astype(jnp.int16)
    a16_ref[...] = a16
    w16_ref[...] = (keys & 0x7FFF).astype(jnp.int16)
    def count16(ind16):
        x = ind16
        n = x.shape[0]
        while n > 16:
            n //= 2
            x = x[:n] + x[n:]
        return jnp.sum(x.astype(jnp.int32), axis=0, keepdims=True)

    def count_ge(ref, pivot_row):
        p16 = pivot_row.astype(jnp.int16)
        return count16((ref[...] >= p16).astype(jnp.int16))

    hi0 = (jnp.max(keys, axis=0, keepdims=True) >> 15) - 32768
    lo0 = jnp.full((1, blk), -32768, jnp.int32)

    def bisect_a(_, lh):
        lo, hi = lh
        mid = lo + ((hi - lo + 1) >> 1)
        ok = count_ge(a16_ref, mid) >= _K
        return jnp.where(ok, mid, lo), jnp.where(ok, hi, mid - 1)

    p_top, _ = lax.fori_loop(0, 16, bisect_a, (lo0, hi0))

    p16 = p_top.astype(jnp.int16)
    cnt_gt_band = count16((a16_ref[...] > p16).astype(jnp.int16))
    kp = _K - cnt_gt_band

    w16_ref[...] = jnp.where(a16_ref[...] == p16, w16_ref[...],
                             jnp.int16(-1))

    def bisect_b(_, lh):
        lo, hi = lh
        mid = lo + ((hi - lo + 1) >> 1)
        ok = count_ge(w16_ref, mid) >= kp
        return jnp.where(ok, mid, lo), jnp.where(ok, hi, mid - 1)

    low15, _ = lax.fori_loop(
        0, 15, bisect_b,
        (jnp.zeros((1, blk), jnp.int32), jnp.full((1, blk), 32767, jnp.int32)))

    t32 = ((p_top + 32768) << 15) | low15
    l16 = low15.astype(jnp.int16)
    cnt_gt_ib = count16((w16_ref[...] > l16).astype(jnp.int16))
    need = kp - cnt_gt_ib

    col16 = lax.broadcasted_iota(jnp.int32, (b, blk), 0).astype(jnp.int16)
    w16_ref[...] = jnp.where(w16_ref[...] == l16, col16, jnp.int16(32767))

    def bisect_c(_, lh):
        lo, hi = lh
        mid = (lo + hi) >> 1
        g = count16((w16_ref[...] <= mid.astype(jnp.int16)).astype(jnp.int16))
        ok = g >= need
        return jnp.where(ok, lo, mid + 1), jnp.where(ok, mid, hi)

    cstar, _ = lax.fori_loop(
        0, 12, bisect_c,
        (jnp.zeros((1, blk), jnp.int32), jnp.full((1, blk), b - 1, jnp.int32)))

    j = pl.program_id(0)
    nblk = pl.num_programs(0)
    bi = jnp.minimum(j, nblk - 2)
    t_scr[pl.ds(bi, 1), :] = t32
    c_scr[pl.ds(bi, 1), :] = cstar

    bj = jnp.maximum(j - 1, 0)
    t_row = t_scr[pl.ds(bj, 1), :]
    c_row = c_scr[pl.ds(bj, 1), :]

    io = lax.broadcasted_iota(jnp.int32, (blk, blk), 0)
    ic = lax.broadcasted_iota(jnp.int32, (blk, blk), 1)
    ident = (io == ic).astype(jnp.float32)

    def col_of(row_i32):
        f = row_i32.astype(jnp.float32)
        col = lax.dot_general(ident, f, (((1,), (1,)), ((), ())),
                              preferred_element_type=jnp.float32)
        return col.astype(jnp.int32)

    t_col = (col_of(t_row >> 16) << 16) | col_of(t_row & 0xFFFF)
    c_col = col_of(c_row)

    adj = lax.dot_general(hb2_ref[...], hall_ref[...],
                          (((1,), (1,)), ((), ())),
                          preferred_element_type=jnp.float32)
    okeys = lax.bitcast_convert_type(adj, jnp.int32)
    ocol = lax.broadcasted_iota(jnp.int32, (blk, b), 1)
    sel = (okeys > t_col) | ((okeys == t_col) & (ocol <= c_col))
    out_ref[...] = jnp.where(sel, adj, 0.0)


def kernel(idx, embn, embhe, W1, b1, W2, b2):
    b = idx.shape[0]
    nhedges, hedim = embhe.shape
    ndim = embn.shape[1]

    nv1_raw = _gather_rows_sc(embn, idx.astype(jnp.int32))

    blk_h = 512
    H = pl.pallas_call(
        _h_body,
        grid=(b // blk_h,),
        in_specs=[
            pl.BlockSpec((blk_h, ndim), lambda i: (i, 0)),
            pl.BlockSpec((W1.shape[0], ndim), lambda i: (0, 0)),
            pl.BlockSpec((1, W1.shape[0]), lambda i: (0, 0)),
            pl.BlockSpec((nhedges, hedim), lambda i: (0, 0)),
            pl.BlockSpec((W2.shape[0], hedim), lambda i: (0, 0)),
            pl.BlockSpec((1, W2.shape[0]), lambda i: (0, 0)),
        ],
        out_specs=pl.BlockSpec((blk_h, nhedges), lambda i: (i, 0)),
        out_shape=jax.ShapeDtypeStruct((b, nhedges), jnp.float32),
        compiler_params=pltpu.CompilerParams(
            dimension_semantics=("parallel",)),
    )(nv1_raw, W1, b1.reshape(1, -1), embhe, W2, b2.reshape(1, -1))

    blk_a = 512
    nblk = b // blk_a
    adj = pl.pallas_call(
        _fused_body,
        grid=(nblk + 1,),
        in_specs=[
            pl.BlockSpec((blk_a, nhedges),
                         lambda j: (jnp.minimum(j, nblk - 1), 0)),
            pl.BlockSpec((blk_a, nhedges),
                         lambda j: (jnp.maximum(j - 1, 0), 0)),
            pl.BlockSpec((b, nhedges), lambda j: (0, 0)),
        ],
        out_specs=pl.BlockSpec((blk_a, b),
                               lambda j: (jnp.maximum(j - 1, 0), 0)),
        out_shape=jax.ShapeDtypeStruct((b, b), jnp.float32),
        scratch_shapes=[pltpu.VMEM((b, blk_a), jnp.int16),
                        pltpu.VMEM((b, blk_a), jnp.int16),
                        pltpu.VMEM((nblk, blk_a), jnp.int32),
                        pltpu.VMEM((nblk, blk_a), jnp.int32)],
        compiler_params=pltpu.CompilerParams(
            vmem_limit_bytes=120 * 1024 * 1024),
    )(H, H, H)

    return adj


# speedup vs baseline: 1.0735x; 1.0735x over previous
import functools

import jax
import jax.numpy as jnp
from jax import lax
from jax.experimental import pallas as pl
from jax.experimental.pallas import tpu as pltpu
from jax.experimental.pallas import tpu_sc as plsc

_ALPHA = 3.0
_K = 20


def _gather_rows_sc(table, idx):
    info = plsc.get_sparse_core_info()
    nc, ns = info.num_cores, info.num_subcores
    nw = nc * ns
    b, d = idx.shape[0], table.shape[1]
    b_per_w = b // nw
    mesh = plsc.VectorSubcoreMesh(core_axis_name="c", subcore_axis_name="s")

    @functools.partial(
        pl.kernel,
        mesh=mesh,
        compiler_params=pltpu.CompilerParams(use_tc_tiling_on_sc=False),
        out_type=jax.ShapeDtypeStruct((b, d), jnp.float32),
        scratch_types=[
            pltpu.VMEM((b_per_w,), jnp.int32),
            pltpu.VMEM((b_per_w, d), jnp.float32),
            pltpu.SemaphoreType.DMA,
        ],
    )
    def gather_kernel(table_hbm, idx_hbm, out_hbm, idx_v, rows_v, sem):
        wid = lax.axis_index("s") * nc + lax.axis_index("c")
        base = wid * b_per_w
        pltpu.sync_copy(idx_hbm.at[pl.ds(base, b_per_w)], idx_v)
        pltpu.async_copy(table_hbm.at[idx_v], rows_v, sem).wait()
        pltpu.sync_copy(rows_v, out_hbm.at[pl.ds(base, b_per_w)])

    return gather_kernel(table, idx)


def _h_body(x_ref, w1_ref, b1_ref, he_ref, w2_ref, b2_ref, h_ref):
    dn = (((1,), (1,)), ((), ()))
    z1 = lax.dot_general(x_ref[...], w1_ref[...], dn,
                         preferred_element_type=jnp.float32)
    nv1 = jnp.tanh(_ALPHA * (z1 + b1_ref[...]))
    z2 = lax.dot_general(he_ref[...], w2_ref[...], dn,
                         preferred_element_type=jnp.float32)
    nv2 = jnp.tanh(_ALPHA * (z2 + b2_ref[...]))
    h0 = lax.dot_general(nv1, nv2, dn, preferred_element_type=jnp.float32)
    h_ref[...] = jnp.maximum(jnp.tanh(_ALPHA * h0), 0.0)


def _fused_body(hb1_ref, hb2_ref, hall_ref, out_ref, a16_ref, w16_ref,
                t_scr, c_scr):
    b, blk = a16_ref.shape
    hb_ref = hb1_ref
    adj_t = lax.dot_general(hall_ref[...], hb_ref[...],
                            (((1,), (1,)), ((), ())),
                            preferred_element_type=jnp.float32)
    keys = lax.bitcast_convert_type(adj_t, jnp.int32)
    a16 = ((keys >> 15) - 32768).astype(jnp.int16)
    a16_ref[...] = a16
    w16_ref[...] = (keys & 0x7FFF).astype(jnp.int16)
    def count16(ind16):
        x = ind16
        n = x.shape[0]
        while n > 16:
            n //= 2
            x = x[:n] + x[n:]
        return jnp.sum(x.astype(jnp.int32), axis=0, keepdims=True)

    def count_ge(ref, pivot_row):
        p16 = pivot_row.astype(jnp.int16)
        return count16((ref[...] >= p16).astype(jnp.int16))

    g32 = keys
    n = g32.shape[0]
    while n > 128:
        n //= 2
        g32 = jnp.maximum(g32[:n], g32[n:])
    ga16 = (g32 >> 15) - 32768
    hi0 = jnp.max(g32, axis=0, keepdims=True) >> 15
    hi0 = hi0 - 32768

    def bisect_g(_, lh):
        lo, hi = lh
        mid = lo + ((hi - lo + 1) >> 1)
        cnt = jnp.sum((ga16 >= mid).astype(jnp.int32), axis=0, keepdims=True)
        ok = cnt >= _K
        return jnp.where(ok, mid, lo), jnp.where(ok, hi, mid - 1)

    lo_g, _ = lax.fori_loop(
        0, 16, bisect_g, (jnp.full((1, blk), -32768, jnp.int32), hi0))

    def bisect_a_cond(lh):
        lo, hi = lh
        return jnp.any(lo < hi)

    def bisect_a(lh):
        lo, hi = lh
        mid = lo + ((hi - lo + 1) >> 1)
        ok = count_ge(a16_ref, mid) >= _K
        return jnp.where(ok, mid, lo), jnp.where(ok, hi, mid - 1)

    p_top, _ = lax.while_loop(bisect_a_cond, bisect_a, (lo_g, hi0))

    p16 = p_top.astype(jnp.int16)
    cnt_gt_band = count16((a16_ref[...] > p16).astype(jnp.int16))
    kp = _K - cnt_gt_band

    w16_ref[...] = jnp.where(a16_ref[...] == p16, w16_ref[...],
                             jnp.int16(-1))

    def bisect_b(_, lh):
        lo, hi = lh
        mid = lo + ((hi - lo + 1) >> 1)
        ok = count_ge(w16_ref, mid) >= kp
        return jnp.where(ok, mid, lo), jnp.where(ok, hi, mid - 1)

    low15, _ = lax.fori_loop(
        0, 15, bisect_b,
        (jnp.zeros((1, blk), jnp.int32), jnp.full((1, blk), 32767, jnp.int32)))

    t32 = ((p_top + 32768) << 15) | low15
    l16 = low15.astype(jnp.int16)
    cnt_gt_ib = count16((w16_ref[...] > l16).astype(jnp.int16))
    need = kp - cnt_gt_ib

    col16 = lax.broadcasted_iota(jnp.int32, (b, blk), 0).astype(jnp.int16)
    w16_ref[...] = jnp.where(w16_ref[...] == l16, col16, jnp.int16(32767))

    def bisect_c(_, lh):
        lo, hi = lh
        mid = (lo + hi) >> 1
        g = count16((w16_ref[...] <= mid.astype(jnp.int16)).astype(jnp.int16))
        ok = g >= need
        return jnp.where(ok, lo, mid + 1), jnp.where(ok, mid, hi)

    cstar, _ = lax.fori_loop(
        0, 12, bisect_c,
        (jnp.zeros((1, blk), jnp.int32), jnp.full((1, blk), b - 1, jnp.int32)))

    j = pl.program_id(0)
    nblk = pl.num_programs(0)
    bi = jnp.minimum(j, nblk - 2)
    t_scr[pl.ds(bi, 1), :] = t32
    c_scr[pl.ds(bi, 1), :] = cstar

    bj = jnp.maximum(j - 1, 0)
    t_row = t_scr[pl.ds(bj, 1), :]
    c_row = c_scr[pl.ds(bj, 1), :]

    io = lax.broadcasted_iota(jnp.int32, (blk, blk), 0)
    ic = lax.broadcasted_iota(jnp.int32, (blk, blk), 1)
    ident = (io == ic).astype(jnp.float32)

    def col_of(row_i32):
        f = row_i32.astype(jnp.float32)
        col = lax.dot_general(ident, f, (((1,), (1,)), ((), ())),
                              preferred_element_type=jnp.float32)
        return col.astype(jnp.int32)

    t_col = (col_of(t_row >> 16) << 16) | col_of(t_row & 0xFFFF)
    c_col = col_of(c_row)

    adj = lax.dot_general(hb2_ref[...], hall_ref[...],
                          (((1,), (1,)), ((), ())),
                          preferred_element_type=jnp.float32)
    okeys = lax.bitcast_convert_type(adj, jnp.int32)
    ocol = lax.broadcasted_iota(jnp.int32, (blk, b), 1)
    sel = (okeys > t_col) | ((okeys == t_col) & (ocol <= c_col))
    out_ref[...] = jnp.where(sel, adj, 0.0)


def kernel(idx, embn, embhe, W1, b1, W2, b2):
    b = idx.shape[0]
    nhedges, hedim = embhe.shape
    ndim = embn.shape[1]

    nv1_raw = _gather_rows_sc(embn, idx.astype(jnp.int32))

    blk_h = 512
    H = pl.pallas_call(
        _h_body,
        grid=(b // blk_h,),
        in_specs=[
            pl.BlockSpec((blk_h, ndim), lambda i: (i, 0)),
            pl.BlockSpec((W1.shape[0], ndim), lambda i: (0, 0)),
            pl.BlockSpec((1, W1.shape[0]), lambda i: (0, 0)),
            pl.BlockSpec((nhedges, hedim), lambda i: (0, 0)),
            pl.BlockSpec((W2.shape[0], hedim), lambda i: (0, 0)),
            pl.BlockSpec((1, W2.shape[0]), lambda i: (0, 0)),
        ],
        out_specs=pl.BlockSpec((blk_h, nhedges), lambda i: (i, 0)),
        out_shape=jax.ShapeDtypeStruct((b, nhedges), jnp.float32),
        compiler_params=pltpu.CompilerParams(
            dimension_semantics=("parallel",)),
    )(nv1_raw, W1, b1.reshape(1, -1), embhe, W2, b2.reshape(1, -1))

    blk_a = 256
    nblk = b // blk_a
    adj = pl.pallas_call(
        _fused_body,
        grid=(nblk + 1,),
        in_specs=[
            pl.BlockSpec((blk_a, nhedges),
                         lambda j: (jnp.minimum(j, nblk - 1), 0)),
            pl.BlockSpec((blk_a, nhedges),
                         lambda j: (jnp.maximum(j - 1, 0), 0)),
            pl.BlockSpec((b, nhedges), lambda j: (0, 0)),
        ],
        out_specs=pl.BlockSpec((blk_a, b),
                               lambda j: (jnp.maximum(j - 1, 0), 0)),
        out_shape=jax.ShapeDtypeStruct((b, b), jnp.float32),
        scratch_shapes=[pltpu.VMEM((b, blk_a), jnp.int16),
                        pltpu.VMEM((b, blk_a), jnp.int16),
                        pltpu.VMEM((nblk, blk_a), jnp.int32),
                        pltpu.VMEM((nblk, blk_a), jnp.int32)],
    )(H, H, H)

    return adj
